# baseline (speedup 1.0000x reference)
import jax
import jax.numpy as jnp
from jax import lax
from jax.experimental import pallas as pl
from jax.experimental.pallas import tpu as pltpu, tpu_sc as plsc

B = 4096
SEQ_LEN = 256
N_BINS = 256
LATENT_DIM = 64

NUM_CORES = 2
NUM_SUBCORES = 16
NW = NUM_CORES * NUM_SUBCORES
BPW = B // NW
BLKB = 8
NBLK = BPW // BLKB
DT = LATENT_DIM // 8
NG = 4


def _body(xs, tok_t, scl_t, pos_t, out,
          tok_v, scl_v, pos_v, obuf, idx_v,
          sem_o0, sem_o1, sem_i):
    sid = lax.axis_index("s")
    wid = sid * NUM_CORES + lax.axis_index("c")
    batch_base = wid * BPW
    sem_o = (sem_o0, sem_o1)

    pltpu.sync_copy(tok_t, tok_v)
    pltpu.sync_copy(scl_t, scl_v)
    pltpu.sync_copy(pos_t, pos_v)
    pltpu.sync_copy(xs.at[pl.ds(batch_base * 4, 4 * BLKB)], idx_v.at[0])

    def blk_body(blk, _):
        h = lax.rem(blk, 2)

        @pl.when(blk > 0)
        def _wait_idx():
            pltpu.make_async_copy(xs.at[pl.ds(0, 4 * BLKB)], idx_v.at[h],
                                  sem_i).wait()

        @pl.when(blk + 1 < NBLK)
        def _prefetch_idx():
            nxt = (batch_base + (blk + 1) * BLKB) * 4
            pltpu.async_copy(xs.at[pl.ds(nxt, 4 * BLKB)], idx_v.at[1 - h], sem_i)

        def q_body(q, _):
            for p in range(2):
                lb = q * 2 + p
                g = blk * BLKB + lb
                batch = batch_base + g

                @pl.when(g >= 2)
                def _drain_out():
                    pltpu.make_async_copy(out.at[0], obuf.at[p], sem_o[p]).wait()

                for st in range(2):
                    for grp in range(2):
                        xt_hi, xt_lo, xs_hi, xs_lo = [], [], [], []
                        for i in range(NG):
                            s0 = (grp * NG + i) * 16
                            xt = idx_v[h, 4 * lb + st, pl.ds(s0, 16)]
                            xsc = idx_v[h, 4 * lb + 2 + st, pl.ds(s0, 16)]
                            xt_hi.append(xt >> 7)
                            xt_lo.append(xt & 127)
                            xs_hi.append(xsc >> 7)
                            xs_lo.append(xsc & 127)

                        def dt_body(dt, _):
                            tt = [xt_hi[i] + 2 * dt for i in range(NG)]
                            ts = [xs_hi[i] + 2 * dt for i in range(NG)]
                            for dr in range(8):
                                rd = jnp.full((16,), dr, jnp.int32)
                                for i in range(NG):
                                    s0 = (grp * NG + i) * 16
                                    tv = plsc.load_gather(
                                        tok_v, [tt[i], rd, xt_lo[i]])
                                    sv = plsc.load_gather(
                                        scl_v, [ts[i], rd, xs_lo[i]])
                                    pv = pos_v[2 * dt + st, dr, pl.ds(s0, 16)]
                                    obuf[p, 8 * dt + dr,
                                         pl.ds(st * 128 + s0, 16)] = tv + sv + pv
                            return 0

                        lax.fori_loop(0, DT, dt_body, 0)

                pltpu.async_copy(obuf.at[p], out.at[batch], sem_o[p])
            return 0

        lax.fori_loop(0, BLKB // 2, q_body, 0)
        return 0

    lax.fori_loop(0, NBLK, blk_body, 0)

    pltpu.make_async_copy(out.at[0], obuf.at[0], sem_o0).wait()
    pltpu.make_async_copy(out.at[0], obuf.at[1], sem_o1).wait()


@jax.jit
def _run(xs, tok_t, scl_t, pos_t):
    mesh = plsc.VectorSubcoreMesh(core_axis_name="c", subcore_axis_name="s")
    kfn = pl.kernel(
        _body,
        out_type=jax.ShapeDtypeStruct((B, LATENT_DIM, SEQ_LEN), jnp.float32),
        mesh=mesh,
        compiler_params=pltpu.CompilerParams(use_tc_tiling_on_sc=True, needs_layout_passes=False),
        scratch_types=[
            pltpu.VMEM((2 * DT, 8, 128), jnp.float32),
            pltpu.VMEM((2 * DT, 8, 128), jnp.float32),
            pltpu.VMEM((2 * DT, 8, 128), jnp.float32),
            pltpu.VMEM((2, LATENT_DIM, SEQ_LEN), jnp.float32),
            pltpu.VMEM((2, 4 * BLKB, 128), jnp.int32),
            pltpu.SemaphoreType.DMA,
            pltpu.SemaphoreType.DMA,
            pltpu.SemaphoreType.DMA,
        ],
    )
    return kfn(xs, tok_t, scl_t, pos_t)


def _tile_t(tab):
    return (tab.T.reshape(DT, 8, 2, 128).transpose(0, 2, 1, 3)
            .reshape(2 * DT, 8, 128))


def kernel(x, token_table, scale_table, pos_table):
    xs = x.reshape(B * 4, 128)
    out = _run(xs, _tile_t(token_table), _tile_t(scale_table),
               _tile_t(pos_table))
    return out.transpose(0, 2, 1)


# speedup vs baseline: 1.0001x; 1.0001x over previous
import jax
import jax.numpy as jnp
from jax import lax
from jax.experimental import pallas as pl
from jax.experimental.pallas import tpu as pltpu, tpu_sc as plsc

B = 4096
SEQ_LEN = 256
N_BINS = 256
LATENT_DIM = 64

NUM_CORES = 2
NUM_SUBCORES = 16
NW = NUM_CORES * NUM_SUBCORES
BPW = B // NW
BLKB = 8
NBLK = BPW // BLKB
DT = LATENT_DIM // 8
NG = 4


def _body(xs, tok_t, scl_t, pos_t, out,
          tok_v, scl_v, pos_v, obuf, idx_v,
          sem_o0, sem_o1, sem_i):
    sid = lax.axis_index("s")
    wid = sid * NUM_CORES + lax.axis_index("c")
    batch_base = wid * BPW
    sem_o = (sem_o0, sem_o1)

    pltpu.sync_copy(tok_t, tok_v)
    pltpu.sync_copy(scl_t, scl_v)
    pltpu.sync_copy(pos_t, pos_v)
    pltpu.sync_copy(xs.at[pl.ds(batch_base * 4, 4 * BLKB)], idx_v.at[0])

    def blk_body(blk, _):
        h = lax.rem(blk, 2)

        @pl.when(blk > 0)
        def _wait_idx():
            pltpu.make_async_copy(xs.at[pl.ds(0, 4 * BLKB)], idx_v.at[h],
                                  sem_i).wait()

        @pl.when(blk + 1 < NBLK)
        def _prefetch_idx():
            nxt = (batch_base + (blk + 1) * BLKB) * 4
            pltpu.async_copy(xs.at[pl.ds(nxt, 4 * BLKB)], idx_v.at[1 - h], sem_i)

        def q_body(q, _):
            for p in range(2):
                lb = q * 2 + p
                g = blk * BLKB + lb
                batch = batch_base + g

                @pl.when(g >= 2)
                def _drain_out():
                    pltpu.make_async_copy(out.at[0], obuf.at[p], sem_o[p]).wait()

                for st in range(2):
                    for grp in range(2):
                        ft, fs = [], []
                        for i in range(NG):
                            s0 = (grp * NG + i) * 16
                            xt = idx_v[h, 4 * lb + st, pl.ds(s0, 16)]
                            xsc = idx_v[h, 4 * lb + 2 + st, pl.ds(s0, 16)]
                            ft.append(((xt >> 7) << 10) + (xt & 127))
                            fs.append(((xsc >> 7) << 10) + (xsc & 127))

                        def dt_body(dt, _):
                            tt = [ft[i] + (dt << 11) for i in range(NG)]
                            ts = [fs[i] + (dt << 11) for i in range(NG)]
                            for dr in range(8):
                                for i in range(NG):
                                    s0 = (grp * NG + i) * 16
                                    tv = plsc.load_gather(
                                        tok_v, [tt[i] + (dr * 128)])
                                    sv = plsc.load_gather(
                                        scl_v, [ts[i] + (dr * 128)])
                                    pv = pos_v[2 * dt + st, dr, pl.ds(s0, 16)]
                                    obuf[p, 8 * dt + dr,
                                         pl.ds(st * 128 + s0, 16)] = tv + sv + pv
                            return 0

                        lax.fori_loop(0, DT, dt_body, 0)

                pltpu.async_copy(obuf.at[p], out.at[batch], sem_o[p])
            return 0

        lax.fori_loop(0, BLKB // 2, q_body, 0)
        return 0

    lax.fori_loop(0, NBLK, blk_body, 0)

    pltpu.make_async_copy(out.at[0], obuf.at[0], sem_o0).wait()
    pltpu.make_async_copy(out.at[0], obuf.at[1], sem_o1).wait()


@jax.jit
def _run(xs, tok_t, scl_t, pos_t):
    mesh = plsc.VectorSubcoreMesh(core_axis_name="c", subcore_axis_name="s")
    kfn = pl.kernel(
        _body,
        out_type=jax.ShapeDtypeStruct((B, LATENT_DIM, SEQ_LEN), jnp.float32),
        mesh=mesh,
        compiler_params=pltpu.CompilerParams(use_tc_tiling_on_sc=True, needs_layout_passes=False),
        scratch_types=[
            pltpu.VMEM((2 * DT * 8 * 128,), jnp.float32),
            pltpu.VMEM((2 * DT * 8 * 128,), jnp.float32),
            pltpu.VMEM((2 * DT, 8, 128), jnp.float32),
            pltpu.VMEM((2, LATENT_DIM, SEQ_LEN), jnp.float32),
            pltpu.VMEM((2, 4 * BLKB, 128), jnp.int32),
            pltpu.SemaphoreType.DMA,
            pltpu.SemaphoreType.DMA,
            pltpu.SemaphoreType.DMA,
        ],
    )
    return kfn(xs, tok_t, scl_t, pos_t)


def _tile_t(tab):
    return (tab.T.reshape(DT, 8, 2, 128).transpose(0, 2, 1, 3)
            .reshape(2 * DT, 8, 128))


def kernel(x, token_table, scale_table, pos_table):
    xs = x.reshape(B * 4, 128)
    out = _run(xs, _tile_t(token_table).reshape(-1),
               _tile_t(scale_table).reshape(-1), _tile_t(pos_table))
    return out.transpose(0, 2, 1)


# speedup vs baseline: 2.5940x; 2.5937x over previous
import jax
import jax.numpy as jnp
from jax import lax
from jax.experimental import pallas as pl
from jax.experimental.pallas import tpu as pltpu, tpu_sc as plsc

B = 4096
SEQ_LEN = 256
N_BINS = 256
LATENT_DIM = 64

NUM_CORES = 2
NUM_SUBCORES = 16
NW = NUM_CORES * NUM_SUBCORES
BPW = B // NW
BLKB = 8
NBLK = BPW // BLKB
DT = LATENT_DIM // 8
NG = 4


def _body(xs, tok_t, scl_t, pos_t, out,
          tok_v, scl_v, pos_v, obuf, idx_v,
          sem_o0, sem_o1, sem_i):
    sid = lax.axis_index("s")
    wid = sid * NUM_CORES + lax.axis_index("c")
    batch_base = wid * BPW
    sem_o = (sem_o0, sem_o1)

    pltpu.sync_copy(tok_t, tok_v)
    pltpu.sync_copy(scl_t, scl_v)
    pltpu.sync_copy(pos_t, pos_v)
    pltpu.sync_copy(xs.at[pl.ds(batch_base * 4, 4 * BLKB)], idx_v.at[0])

    def blk_body(blk, _):
        h = lax.rem(blk, 2)

        @pl.when(blk > 0)
        def _wait_idx():
            pltpu.make_async_copy(xs.at[pl.ds(0, 4 * BLKB)], idx_v.at[h],
                                  sem_i).wait()

        @pl.when(blk + 1 < NBLK)
        def _prefetch_idx():
            nxt = (batch_base + (blk + 1) * BLKB) * 4
            pltpu.async_copy(xs.at[pl.ds(nxt, 4 * BLKB)], idx_v.at[1 - h], sem_i)

        def q_body(q, _):
            for p in range(2):
                lb = q * 2 + p
                g = blk * BLKB + lb
                batch = batch_base + g

                @pl.when(g >= 2)
                def _drain_out():
                    pltpu.make_async_copy(out.at[0], obuf.at[p], sem_o[p]).wait()

                for st in range(2):
                    for grp in range(2):
                        ft, fs = [], []
                        for i in range(NG):
                            s0 = (grp * NG + i) * 16
                            xt = idx_v[h, 4 * lb + st, pl.ds(s0, 16)]
                            xsc = idx_v[h, 4 * lb + 2 + st, pl.ds(s0, 16)]
                            ft.append(((xt >> 7) << 10) + (xt & 127))
                            fs.append(((xsc >> 7) << 10) + (xsc & 127))

                        def dt_body(dt, _):
                            base = dt * 2048
                            for dr in range(8):
                                off = pl.ds(base + dr * 128, 1024)
                                tv = [plsc.load_gather(tok_v.at[off], [ft[i]])
                                      for i in range(NG)]
                                sv = [plsc.load_gather(scl_v.at[off], [fs[i]])
                                      for i in range(NG)]
                                pv = [pos_v[2 * dt + st, dr,
                                            pl.ds((grp * NG + i) * 16, 16)]
                                      for i in range(NG)]
                                for i in range(NG):
                                    s0 = (grp * NG + i) * 16
                                    obuf[p, 8 * dt + dr,
                                         pl.ds(st * 128 + s0, 16)] = (
                                             tv[i] + sv[i] + pv[i])
                            return 0

                        lax.fori_loop(0, DT, dt_body, 0)

                pltpu.async_copy(obuf.at[p], out.at[batch], sem_o[p])
            return 0

        lax.fori_loop(0, BLKB // 2, q_body, 0)
        return 0

    lax.fori_loop(0, NBLK, blk_body, 0)

    pltpu.make_async_copy(out.at[0], obuf.at[0], sem_o0).wait()
    pltpu.make_async_copy(out.at[0], obuf.at[1], sem_o1).wait()


@jax.jit
def _run(xs, tok_t, scl_t, pos_t):
    mesh = plsc.VectorSubcoreMesh(core_axis_name="c", subcore_axis_name="s")
    kfn = pl.kernel(
        _body,
        out_type=jax.ShapeDtypeStruct((B, LATENT_DIM, SEQ_LEN), jnp.float32),
        mesh=mesh,
        compiler_params=pltpu.CompilerParams(use_tc_tiling_on_sc=True, needs_layout_passes=False),
        scratch_types=[
            pltpu.VMEM((2 * DT * 8 * 128,), jnp.float32),
            pltpu.VMEM((2 * DT * 8 * 128,), jnp.float32),
            pltpu.VMEM((2 * DT, 8, 128), jnp.float32),
            pltpu.VMEM((2, LATENT_DIM, SEQ_LEN), jnp.float32),
            pltpu.VMEM((2, 4 * BLKB, 128), jnp.int32),
            pltpu.SemaphoreType.DMA,
            pltpu.SemaphoreType.DMA,
            pltpu.SemaphoreType.DMA,
        ],
    )
    return kfn(xs, tok_t, scl_t, pos_t)


def _tile_t(tab):
    return (tab.T.reshape(DT, 8, 2, 128).transpose(0, 2, 1, 3)
            .reshape(2 * DT, 8, 128))


def kernel(x, token_table, scale_table, pos_table):
    xs = x.reshape(B * 4, 128)
    out = _run(xs, _tile_t(token_table).reshape(-1),
               _tile_t(scale_table).reshape(-1), _tile_t(pos_table))
    return out.transpose(0, 2, 1)
